# baseline (speedup 1.0000x reference)
import functools

import jax
import jax.numpy as jnp
from jax import lax
from jax.experimental import pallas as pl
from jax.experimental.pallas import tpu as pltpu
from jax.experimental.pallas import tpu_sc as plsc

B = 128 * 16 * 16 * 16
D = 32
NC = 2
NS = 16
NW = NC * NS
BPW = B // NW
ROW = 128
NROWS = BPW // ROW
NB = 8
NSTEP = NROWS // NB

_mesh = plsc.VectorSubcoreMesh(core_axis_name="c", subcore_axis_name="s")


@functools.partial(
    pl.kernel,
    mesh=_mesh,
    compiler_params=pltpu.CompilerParams(use_tc_tiling_on_sc=False),
    out_type=jax.ShapeDtypeStruct((B, D), jnp.float32),
    scratch_types=[
        pltpu.VMEM((NROWS, ROW), jnp.int32),
        pltpu.VMEM((2, NB * ROW, D), jnp.float32),
        pltpu.SemaphoreType.DMA,
        pltpu.SemaphoreType.DMA,
    ],
)
def _gather(idx_hbm, table_hbm, out_hbm, idx_v, rbuf, gsem, osem):
    wid = lax.axis_index("s") * NC + lax.axis_index("c")
    base = wid * BPW
    pltpu.sync_copy(idx_hbm.at[wid], idx_v)

    def fire_gathers(s, buf):
        hs = []
        for b in range(NB):
            j = s * NB + b
            hs.append(
                pltpu.async_copy(
                    table_hbm.at[idx_v.at[j]], buf.at[pl.ds(b * ROW, ROW)], gsem
                )
            )
        return hs

    gh = fire_gathers(0, rbuf.at[0])
    wh = {}
    for s in range(NSTEP):
        cur = rbuf.at[s % 2]
        if s + 1 < NSTEP:
            if s >= 1:
                wh[s - 1].wait()
            nxt_gh = fire_gathers(s + 1, rbuf.at[(s + 1) % 2])
        for h in gh:
            h.wait()
        wh[s] = pltpu.async_copy(
            cur, out_hbm.at[pl.ds(base + s * NB * ROW, NB * ROW)], osem
        )
        if s + 1 < NSTEP:
            gh = nxt_gh
    wh[NSTEP - 2].wait()
    wh[NSTEP - 1].wait()


def kernel(inputs, table):
    idx = inputs.reshape(NW, NROWS, ROW)
    out = _gather(idx, table)
    b, x, y = inputs.shape[0], inputs.shape[1], inputs.shape[2]
    return out.reshape(b, x, y, -1)


# speedup vs baseline: 1.0036x; 1.0036x over previous
import functools

import jax
import jax.numpy as jnp
from jax import lax
from jax.experimental import pallas as pl
from jax.experimental.pallas import tpu as pltpu
from jax.experimental.pallas import tpu_sc as plsc

B = 128 * 16 * 16 * 16
D = 32
NC = 2
NS = 16
NW = NC * NS
BPW = B // NW
ROW = 128
NROWS = BPW // ROW
NB = 8
NSTEP = NROWS // NB

_mesh = plsc.VectorSubcoreMesh(core_axis_name="c", subcore_axis_name="s")


@functools.partial(
    pl.kernel,
    mesh=_mesh,
    compiler_params=pltpu.CompilerParams(use_tc_tiling_on_sc=False),
    out_type=jax.ShapeDtypeStruct((B, D), jnp.float32),
    scratch_types=[
        pltpu.VMEM((BPW,), jnp.int32),
        pltpu.VMEM((2, NB * ROW, D), jnp.float32),
        pltpu.SemaphoreType.DMA,
        pltpu.SemaphoreType.DMA,
    ],
)
def _gather(idx_hbm, table_hbm, out_hbm, idx_v, rbuf, gsem, osem):
    wid = lax.axis_index("s") * NC + lax.axis_index("c")
    pltpu.sync_copy(idx_hbm.at[pl.ds(wid * BPW, BPW)], idx_v)

    def fire_gathers(s, buf):
        hs = []
        for b in range(NB):
            j = s * NB + b
            hs.append(
                pltpu.async_copy(
                    table_hbm.at[idx_v.at[pl.ds(j * ROW, ROW)]],
                    buf.at[pl.ds(b * ROW, ROW)],
                    gsem,
                )
            )
        return hs

    gh = fire_gathers(0, rbuf.at[0])
    wh = {}
    for s in range(NSTEP):
        cur = rbuf.at[s % 2]
        if s + 1 < NSTEP:
            if s >= 1:
                wh[s - 1].wait()
            nxt_gh = fire_gathers(s + 1, rbuf.at[(s + 1) % 2])
        for h in gh:
            h.wait()
        wh[s] = pltpu.async_copy(
            cur, out_hbm.at[pl.ds(wid * BPW + s * NB * ROW, NB * ROW)], osem
        )
        if s + 1 < NSTEP:
            gh = nxt_gh
    wh[NSTEP - 2].wait()
    wh[NSTEP - 1].wait()


def kernel(inputs, table):
    out = _gather(inputs.reshape(-1), table)
    b, x, y = inputs.shape[0], inputs.shape[1], inputs.shape[2]
    return out.reshape(b, x, y, -1)
